# baseline (speedup 1.0000x reference)
import functools

import numpy as np
import jax
import jax.numpy as jnp
from jax.experimental import pallas as pl
from jax.experimental.pallas import tpu as pltpu


def _fused_kernel(qc_ref, p_ref, t1_ref, b1_ref,
                  c1_ref, t2_ref, b2_ref, c2_ref, w1_ref, v1_ref, w2_ref,
                  v2_ref, o_ref, *, num_class, bsz):
    f32, bf16 = jnp.float32, jnp.bfloat16
    hq = 16

    def interleave(m):
        qq = qc_ref[:, 51 * m:51 * (m + 1), :]
        cat = jnp.concatenate([qq[:, 0:17], qq[:, 17:34], qq[:, 34:51]],
                              axis=2).reshape(bsz * 17, -1)
        sl = jnp.dot(cat, p_ref[...], preferred_element_type=f32)
        return sl.astype(bf16).reshape(bsz, 17, -1)

    q = (interleave(0), interleave(1), interleave(2), interleave(3))

    def lhs_phase(c):
        parts = []
        for kh in range(5):
            m, o = (c + kh) % 4, (c + kh) // 4
            parts.append(q[m][:, o:o + hq, :])
        return jnp.concatenate(parts, axis=2).reshape(bsz * hq, -1)

    y = [jnp.dot(lhs_phase(c), t1_ref[...], preferred_element_type=f32)
         for c in range(4)]
    p_e = jnp.maximum(jnp.maximum(y[0], y[1]) + b1_ref[...], 0.0)
    p_o = jnp.maximum(jnp.maximum(y[2], y[3]) + b1_ref[...], 0.0)

    def premax(p, cshift):
        return jnp.maximum(p, jnp.concatenate(
            [p[:, cshift:], p[:, :cshift]], axis=1))
    pm_e = premax(p_e, 16).astype(bf16)
    pm_o = premax(p_o, 16).astype(bf16)
    cc_e = jnp.dot(pm_e, c1_ref[...], preferred_element_type=f32)
    cc_o = jnp.dot(pm_o, c1_ref[...], preferred_element_type=f32)
    s2 = cc_e.shape[1]
    p1_e = cc_e.astype(bf16).reshape(bsz, hq, s2)
    p1_o = cc_o.astype(bf16).reshape(bsz, hq, s2)

    z1 = jnp.zeros((bsz, 1, s2), bf16)
    ce = jnp.concatenate([z1, p1_e, z1], axis=1)
    co = jnp.concatenate([z1, p1_o, z1], axis=1)
    lhs2_e = jnp.concatenate(
        [ce[:, 0:16], co[:, 0:16], ce[:, 1:17], co[:, 1:17], ce[:, 2:18]],
        axis=2).reshape(bsz * hq, -1)
    lhs2_o = jnp.concatenate(
        [co[:, 0:16], ce[:, 1:17], co[:, 1:17], ce[:, 2:18], co[:, 2:18]],
        axis=2).reshape(bsz * hq, -1)
    y2_e = jnp.dot(lhs2_e, t2_ref[...], preferred_element_type=f32)
    y2_o = jnp.dot(lhs2_o, t2_ref[...], preferred_element_type=f32)
    p2r = jnp.maximum(jnp.maximum(y2_e, y2_o) + b2_ref[...], 0.0)

    pm2 = premax(p2r, 32).astype(bf16)
    cc2 = jnp.dot(pm2, c2_ref[...], preferred_element_type=f32)
    f2 = cc2.shape[1]
    p2 = cc2.astype(bf16).reshape(bsz, hq, f2)

    acc = jnp.broadcast_to(v1_ref[...], (bsz, v1_ref.shape[1])).astype(f32)
    for yy in range(hq):
        acc = acc + jnp.dot(p2[:, yy, :], w1_ref[yy * f2:(yy + 1) * f2, :],
                            preferred_element_type=f32)
    hid = jnp.maximum(acc, 0.0)

    logits = jnp.dot(hid.astype(bf16), w2_ref[...],
                     preferred_element_type=f32) + v2_ref[...]
    col = jax.lax.broadcasted_iota(jnp.int32, logits.shape, 1)
    logits = jnp.where(col < num_class, logits, -1e30)
    m = jnp.max(logits, axis=-1, keepdims=True)
    sh = logits - m
    lse = jnp.log(jnp.sum(jnp.exp(sh), axis=-1, keepdims=True))
    o_ref[...] = (sh - lse)[:, :num_class].astype(o_ref.dtype)


def kernel(x, T1, b1, R1, C1, T2, b2, R2, C2, fc1_w, fc1_b, fc2_w, fc2_b):
    del R1, R2
    N, cin, H, W = x.shape
    pad, ksize = 2, 5
    slab1 = T1.shape[0] // ksize
    Hp, Wp = H + 2 * pad, W + 2 * pad
    npad = fc2_w.shape[1]
    num_class = 100
    bsz = 64 if N % 64 == 0 else (8 if N % 8 == 0 else 1)

    lanes = 128
    xpl = jnp.pad(x, ((0, 0), (0, 0), (pad, pad),
                      (pad, lanes - W - pad))).astype(jnp.bfloat16)
    hq4 = (Hp + 3) // 4
    qcat = jnp.concatenate(
        [xpl[:, :, m::4, :].reshape(N, cin * hq4, lanes) for m in range(4)],
        axis=1)

    p_np = np.zeros((cin * lanes, slab1), np.float32)
    for c in range(cin):
        for w in range(Wp):
            p_np[c * lanes + w, w * cin + c] = 1.0
    P = jnp.asarray(p_np, jnp.bfloat16)

    c1h = C1[:, :C1.shape[1] // 2].astype(jnp.bfloat16)
    c2h = C2[:, :C2.shape[1] // 2].astype(jnp.bfloat16)
    consts = (P, T1, b1, c1h, T2, b2, c2h, fc1_w, fc1_b, fc2_w, fc2_b)

    def const_spec(a):
        nd = a.ndim
        return pl.BlockSpec(a.shape, lambda n, _nd=nd: (0,) * _nd)

    out = pl.pallas_call(
        functools.partial(_fused_kernel, num_class=num_class, bsz=bsz),
        out_shape=jax.ShapeDtypeStruct((N, num_class), jnp.float32),
        grid=(N // bsz,),
        in_specs=[pl.BlockSpec((bsz, 4 * cin * hq4, lanes), lambda n: (n, 0, 0))]
                 + [const_spec(a) for a in consts],
        out_specs=pl.BlockSpec((bsz, num_class), lambda n: (n, 0)),
        compiler_params=pltpu.CompilerParams(dimension_semantics=("parallel",)),
    )(qcat, *consts)
    return out


# speedup vs baseline: 1.0801x; 1.0801x over previous
import functools

import numpy as np
import jax
import jax.numpy as jnp
from jax.experimental import pallas as pl
from jax.experimental.pallas import tpu as pltpu


def _fused_kernel(q0_ref, q1_ref, q2_ref, q3_ref, p_ref, t1_ref, b1_ref,
                  c1_ref, t2_ref, b2_ref, c2_ref, w1_ref, v1_ref, w2_ref,
                  v2_ref, o_ref, *, num_class, bsz):
    f32, bf16 = jnp.float32, jnp.bfloat16
    hq = 16

    def interleave(qr):
        qq = qr[...]
        cat = jnp.concatenate([qq[:, 0:17], qq[:, 17:34], qq[:, 34:51]],
                              axis=2).reshape(bsz * 17, -1)
        sl = jnp.dot(cat, p_ref[...], preferred_element_type=f32)
        return sl.astype(bf16).reshape(bsz, 17, -1)

    q = (interleave(q0_ref), interleave(q1_ref),
         interleave(q2_ref), interleave(q3_ref))

    def lhs_phase(c):
        parts = []
        for kh in range(5):
            m, o = (c + kh) % 4, (c + kh) // 4
            parts.append(q[m][:, o:o + hq, :])
        return jnp.concatenate(parts, axis=2).reshape(bsz * hq, -1)

    y = [jnp.dot(lhs_phase(c), t1_ref[...], preferred_element_type=f32)
         for c in range(4)]
    p_e = jnp.maximum(jnp.maximum(y[0], y[1]) + b1_ref[...], 0.0)
    p_o = jnp.maximum(jnp.maximum(y[2], y[3]) + b1_ref[...], 0.0)

    def premax(p, cshift):
        return jnp.maximum(p, jnp.concatenate(
            [p[:, cshift:], p[:, :cshift]], axis=1))
    pm_e = premax(p_e, 16).astype(bf16)
    pm_o = premax(p_o, 16).astype(bf16)
    cc_e = jnp.dot(pm_e, c1_ref[...], preferred_element_type=f32)
    cc_o = jnp.dot(pm_o, c1_ref[...], preferred_element_type=f32)
    s2 = cc_e.shape[1]
    p1_e = cc_e.astype(bf16).reshape(bsz, hq, s2)
    p1_o = cc_o.astype(bf16).reshape(bsz, hq, s2)

    z1 = jnp.zeros((bsz, 1, s2), bf16)
    ce = jnp.concatenate([z1, p1_e, z1], axis=1)
    co = jnp.concatenate([z1, p1_o, z1], axis=1)
    lhs2_e = jnp.concatenate(
        [ce[:, 0:16], co[:, 0:16], ce[:, 1:17], co[:, 1:17], ce[:, 2:18]],
        axis=2).reshape(bsz * hq, -1)
    lhs2_o = jnp.concatenate(
        [co[:, 0:16], ce[:, 1:17], co[:, 1:17], ce[:, 2:18], co[:, 2:18]],
        axis=2).reshape(bsz * hq, -1)
    y2_e = jnp.dot(lhs2_e, t2_ref[...], preferred_element_type=f32)
    y2_o = jnp.dot(lhs2_o, t2_ref[...], preferred_element_type=f32)
    p2r = jnp.maximum(jnp.maximum(y2_e, y2_o) + b2_ref[...], 0.0)

    pm2 = premax(p2r, 32).astype(bf16)
    cc2 = jnp.dot(pm2, c2_ref[...], preferred_element_type=f32)
    f2 = cc2.shape[1]
    p2 = cc2.astype(bf16).reshape(bsz, hq, f2)

    acc = jnp.broadcast_to(v1_ref[...], (bsz, v1_ref.shape[1])).astype(f32)
    for yy in range(hq):
        acc = acc + jnp.dot(p2[:, yy, :], w1_ref[yy * f2:(yy + 1) * f2, :],
                            preferred_element_type=f32)
    hid = jnp.maximum(acc, 0.0)

    logits = jnp.dot(hid.astype(bf16), w2_ref[...],
                     preferred_element_type=f32) + v2_ref[...]
    col = jax.lax.broadcasted_iota(jnp.int32, logits.shape, 1)
    logits = jnp.where(col < num_class, logits, -1e30)
    m = jnp.max(logits, axis=-1, keepdims=True)
    sh = logits - m
    lse = jnp.log(jnp.sum(jnp.exp(sh), axis=-1, keepdims=True))
    o_ref[...] = (sh - lse)[:, :num_class].astype(o_ref.dtype)


def kernel(x, T1, b1, R1, C1, T2, b2, R2, C2, fc1_w, fc1_b, fc2_w, fc2_b):
    del R1, R2
    N, cin, H, W = x.shape
    pad, ksize = 2, 5
    slab1 = T1.shape[0] // ksize
    Hp, Wp = H + 2 * pad, W + 2 * pad
    npad = fc2_w.shape[1]
    num_class = 100
    bsz = 64 if N % 64 == 0 else (8 if N % 8 == 0 else 1)

    lanes = 128
    xpl = jnp.pad(x, ((0, 0), (0, 0), (pad, pad),
                      (pad, lanes - W - pad))).astype(jnp.bfloat16)
    hq4 = (Hp + 3) // 4
    qs = [xpl[:, :, m::4, :].reshape(N, cin * hq4, lanes)
          for m in range(4)]

    p_np = np.zeros((cin * lanes, slab1), np.float32)
    for c in range(cin):
        for w in range(Wp):
            p_np[c * lanes + w, w * cin + c] = 1.0
    P = jnp.asarray(p_np, jnp.bfloat16)

    c1h = C1[:, :C1.shape[1] // 2].astype(jnp.bfloat16)
    c2h = C2[:, :C2.shape[1] // 2].astype(jnp.bfloat16)
    consts = (P, T1, b1, c1h, T2, b2, c2h, fc1_w, fc1_b, fc2_w, fc2_b)

    def const_spec(a):
        nd = a.ndim
        return pl.BlockSpec(a.shape, lambda n, _nd=nd: (0,) * _nd)

    out = pl.pallas_call(
        functools.partial(_fused_kernel, num_class=num_class, bsz=bsz),
        out_shape=jax.ShapeDtypeStruct((N, num_class), jnp.float32),
        grid=(N // bsz,),
        in_specs=[pl.BlockSpec((bsz, cin * hq4, lanes), lambda n: (n, 0, 0))] * 4
                 + [const_spec(a) for a in consts],
        out_specs=pl.BlockSpec((bsz, num_class), lambda n: (n, 0)),
        compiler_params=pltpu.CompilerParams(dimension_semantics=("parallel",)),
    )(*qs, *consts)
    return out
